# Initial kernel scaffold; baseline (speedup 1.0000x reference)
import jax
import jax.numpy as jnp
from jax.experimental import pallas as pl


def kernel(x_field, mesh_x, boundary, edge_attr, edge_index, params):
    raise NotImplementedError("write your pallas kernel here")


# speedup vs baseline: 3.3175x; 3.3175x over previous
import functools

import jax
import jax.numpy as jnp
from jax import lax
from jax.experimental import pallas as pl
from jax.experimental.pallas import tpu as pltpu
from jax.experimental.pallas import tpu_sc as plsc

N = 10000
E = 160000
NF = 8
NB = 4
NM = 128
NE = 4

NPAD = 10240
SUB = 128
EPT = 5120
EPAD = 32 * EPT
NS = EPT // SUB
ZR = NPAD // 16


@functools.lru_cache(maxsize=None)
def _make_agg(d, rows_in):
    mesh = plsc.VectorSubcoreMesh(core_axis_name="c", subcore_axis_name="s")

    @functools.partial(
        pl.kernel,
        out_type=jax.ShapeDtypeStruct((2, NPAD, d), jnp.float32),
        mesh=mesh,
        scratch_types=[
            pltpu.VMEM((NS, SUB), jnp.int32),
            pltpu.VMEM((NS, SUB), jnp.int32),
            pltpu.VMEM((SUB, d), jnp.float32),
            pltpu.VMEM_SHARED((NPAD, d), jnp.float32),
            pltpu.SemaphoreType.DMA,
        ],
        compiler_params=pltpu.CompilerParams(use_tc_tiling_on_sc=False),
    )
    def agg(table, srcw, dstw, zrows, out, idx_s, idx_d, rows, acc, sem):
        c = lax.axis_index("c")
        s = lax.axis_index("s")
        pltpu.sync_copy(zrows, acc.at[pl.ds(s * ZR, ZR)])
        pltpu.sync_copy(srcw.at[c, s], idx_s)
        pltpu.sync_copy(dstw.at[c, s], idx_d)
        plsc.subcore_barrier()

        def body(j, carry):
            pltpu.async_copy(table.at[idx_s.at[j]], rows, sem).wait()
            pltpu.sync_copy(rows, acc.at[idx_d.at[j]], add=True)
            return carry

        lax.fori_loop(0, NS, body, 0)
        plsc.subcore_barrier()
        pltpu.sync_copy(acc.at[pl.ds(s * ZR, ZR)], out.at[c, pl.ds(s * ZR, ZR)])

    return agg


def _tc_stage(inputs, body_fn, out_widths, rows=NPAD, bm=1024):
    grid = (rows // bm,)
    in_specs = []
    for a in inputs:
        if a.shape[0] == rows:
            in_specs.append(pl.BlockSpec((bm, a.shape[1]), lambda i: (i, 0)))
        else:
            in_specs.append(pl.BlockSpec(a.shape, lambda i: (0, 0)))
    out_shape = tuple(jax.ShapeDtypeStruct((rows, w), jnp.float32)
                      for w in out_widths)
    out_specs = tuple(pl.BlockSpec((bm, w), lambda i: (i, 0))
                      for w in out_widths)

    def kern(*refs):
        ins = refs[:len(inputs)]
        outs = refs[len(inputs):]
        vals = body_fn(*[r[...] for r in ins])
        if not isinstance(vals, tuple):
            vals = (vals,)
        for o, v in zip(outs, vals):
            o[...] = v

    res = pl.pallas_call(
        kern, grid=grid, in_specs=in_specs, out_specs=out_specs,
        out_shape=out_shape)(*inputs)
    return res


def _dot(x, w):
    return jnp.dot(x, w, preferred_element_type=jnp.float32)


def kernel(x_field, mesh_x, boundary, edge_attr, edge_index, params):
    p = params
    f32 = jnp.float32

    def padN(a):
        return jnp.pad(a, ((0, NPAD - N), (0, 0)))

    xf = padN(x_field)
    mx = padN(mesh_x)
    bd = padN(boundary)
    ea = jnp.pad(edge_attr, ((0, EPAD - E), (0, 0)))
    srcp = jnp.pad(edge_index[0], (0, EPAD - E))
    dstp = jnp.pad(edge_index[1], (0, EPAD - E), constant_values=NPAD - 1)
    src4 = srcp.reshape(2, 16, NS, SUB)
    dst4 = dstp.reshape(2, 16, NS, SUB)
    eidx4 = jnp.arange(EPAD, dtype=jnp.int32).reshape(2, 16, NS, SUB)
    zeros = {d: jnp.zeros((ZR, d), f32) for d in (16, 32, 64, 128)}

    def agg(table, idx=None):
        d = table.shape[1]
        i4 = src4 if idx is None else idx
        return _make_agg(d, table.shape[0])(table, i4, dst4, zeros[d])

    def b2(name):
        return p[name].reshape(1, -1)

    def bpad(name, d):
        b = p[name]
        return jnp.pad(b, (0, d - b.shape[0])).reshape(1, -1)

    def wpad(name, d):
        w = p[name]
        return jnp.pad(w, ((0, 0), (0, d - w.shape[1])))

    r = jax.nn.relu

    w_mesh_n = p["W_mesh"][:NM]
    w_mesh_e = p["W_mesh"][NM:]
    (ym,) = _tc_stage([mx, w_mesh_n], lambda x, w: _dot(x, w), (NM,))
    (t_edges,) = _tc_stage([ea, w_mesh_e], lambda x, w: _dot(x, w), (NM,),
                           rows=EPAD, bm=2048)
    am = agg(ym)
    at = agg(t_edges, eidx4)

    (m, yu1) = _tc_stage(
        [am[0], am[1], at[0], at[1], b2("b_mesh"), p["W_u1"]],
        lambda a0, a1, a2, a3, b, w: (
            lambda mm: (mm, _dot(mm, w)))(r(a0 + a1 + a2 + a3 + b)),
        (NM, NM))

    a = agg(yu1)
    (u1, yu2) = _tc_stage(
        [a[0], a[1], b2("b_u1"), m, p["W_u2"]],
        lambda a0, a1, b, res, w: (
            lambda u: (u, _dot(u, w)))(r(a0 + a1 + b) + res),
        (NM, NM))
    a = agg(yu2)
    (u2, yu3) = _tc_stage(
        [a[0], a[1], b2("b_u2"), u1, p["W_u3"]],
        lambda a0, a1, b, res, w: (
            lambda u: (u, _dot(u, w)))(r(a0 + a1 + b) + res),
        (NM, NM))
    a = agg(yu3)
    wd10 = p["W_d10"]
    (yd10,) = _tc_stage(
        [a[0], a[1], b2("b_u3"), u2, xf, bd, wd10[:NF], wd10[NF:NF + NB],
         wd10[NF + NB:]],
        lambda a0, a1, b, res, x, bdv, w1, w2, w3: (
            lambda u: _dot(x, w1) + _dot(bdv, w2) + _dot(u, w3))(
                r(a0 + a1 + b) + res),
        (64,))

    a = agg(yd10)
    (d0, yd11) = _tc_stage(
        [a[0], a[1], b2("b_d10"), p["W_d11"]],
        lambda a0, a1, b, w: (lambda x: (x, _dot(x, w)))(r(a0 + a1 + b)),
        (64, 64))
    a = agg(yd11)
    (yd12,) = _tc_stage(
        [a[0], a[1], b2("b_d11"), p["W_d12"]],
        lambda a0, a1, b, w: _dot(r(a0 + a1 + b), w),
        (64,))
    a = agg(yd12)
    (d2,) = _tc_stage(
        [a[0], a[1], b2("b_d12"), d0],
        lambda a0, a1, b, res: r(a0 + a1 + b) + res,
        (64,))

    a = agg(d2)
    (e0, ye1) = _tc_stage(
        [a[0], a[1], p["W_d20"], b2("b_d20"), p["W_d21"]],
        lambda a0, a1, w0, b, w: (
            lambda x: (x, _dot(x, w)))(r(_dot(a0 + a1, w0) + b)),
        (NM, NM))
    a = agg(ye1)
    (ye2,) = _tc_stage(
        [a[0], a[1], b2("b_d21"), p["W_d22"]],
        lambda a0, a1, b, w: _dot(r(a0 + a1 + b), w),
        (NM,))
    a = agg(ye2)
    (yf0,) = _tc_stage(
        [a[0], a[1], b2("b_d22"), e0, p["W_d30"]],
        lambda a0, a1, b, res, w: _dot(r(a0 + a1 + b) + res, w),
        (NM,))

    a = agg(yf0)
    (yf1,) = _tc_stage(
        [a[0], a[1], b2("b_d30"), p["W_d31"]],
        lambda a0, a1, b, w: _dot(r(a0 + a1 + b), w),
        (64,))
    a = agg(yf1)
    (yf2,) = _tc_stage(
        [a[0], a[1], b2("b_d31"), p["W_d32"]],
        lambda a0, a1, b, w: _dot(r(a0 + a1 + b), w),
        (32,))
    a = agg(yf2)
    (yfd,) = _tc_stage(
        [a[0], a[1], b2("b_d32"), wpad("W_fdot", 16)],
        lambda a0, a1, b, w: _dot(r(a0 + a1 + b), w),
        (16,))
    a = agg(yfd)
    (fdot,) = _tc_stage(
        [a[0], a[1], bpad("b_fdot", 16)],
        lambda a0, a1, b: a0 + a1 + b,
        (16,))

    a = agg(fdot)
    (i0, yi1) = _tc_stage(
        [a[0], a[1], jnp.pad(p["W_i10"], ((0, 8), (0, 0))), b2("b_i10"),
         p["W_i11"]],
        lambda a0, a1, w0, b, w: (
            lambda x: (x, _dot(x, w)))(r(_dot(a0 + a1, w0) + b)),
        (64, 64))
    a = agg(yi1)
    (yi2,) = _tc_stage(
        [a[0], a[1], b2("b_i11"), p["W_i12"]],
        lambda a0, a1, b, w: _dot(r(a0 + a1 + b), w),
        (64,))
    a = agg(yi2)
    (yio,) = _tc_stage(
        [a[0], a[1], b2("b_i12"), i0, wpad("W_iout", 16)],
        lambda a0, a1, b, res, w: _dot(r(a0 + a1 + b) + res, w),
        (16,))
    a = agg(yio)
    (out,) = _tc_stage(
        [a[0], a[1], bpad("b_iout", 16), xf],
        lambda a0, a1, b, x: x + (a0 + a1 + b)[:, :NF],
        (NF,))

    return out[:N]
